# Initial kernel scaffold; baseline (speedup 1.0000x reference)
import functools
import jax
import jax.numpy as jnp
from jax import lax
from jax.experimental import pallas as pl
from jax.experimental.pallas import tpu as pltpu
from jax.experimental.pallas import tpu_sc as plsc

NC, NS = 2, 16
NW = NC * NS
D = 64
CHUNK = 128


@functools.partial(jax.jit, static_argnames=("n_rows",))
def _gather_rows(idx2d, table, n_rows):
    n_chunks_total = idx2d.shape[0]
    n_chunks = n_chunks_total // NW
    rows_per_w = n_chunks * CHUNK

    mesh = plsc.VectorSubcoreMesh(
        core_axis_name="c", subcore_axis_name="s",
        num_cores=NC, num_subcores=NS)

    @functools.partial(
        pl.kernel,
        out_type=jax.ShapeDtypeStruct((n_rows, D), jnp.float32),
        mesh=mesh,
        scratch_types=[
            pltpu.VMEM((n_chunks, CHUNK), jnp.int32),
            pltpu.VMEM((CHUNK, D), jnp.float32),
            pltpu.SemaphoreType.DMA,
        ],
    )
    def k(idx_hbm, table_hbm, out_hbm, idx_v, rows_v, sem):
        wid = lax.axis_index("s") * NC + lax.axis_index("c")
        pltpu.sync_copy(idx_hbm.at[pl.ds(wid * n_chunks, n_chunks)], idx_v)
        row_base = wid * rows_per_w

        @pl.loop(0, n_chunks)
        def body(j):
            pltpu.async_copy(table_hbm.at[idx_v.at[j]], rows_v, sem).wait()
            pltpu.sync_copy(rows_v, out_hbm.at[pl.ds(row_base + j * CHUNK, CHUNK)])

    return k(idx2d, table)


def kernel(token_ids, embeddings):
    b, s = token_ids.shape
    n_rows = b * s
    idx2d = token_ids.astype(jnp.int32).reshape(n_rows // CHUNK, CHUNK)
    out = _gather_rows(idx2d, embeddings, n_rows)
    return out.reshape(b, s, D)


# speedup vs baseline: 1.6855x; 1.6855x over previous
import functools
import jax
import jax.numpy as jnp
from jax import lax
from jax.experimental import pallas as pl
from jax.experimental.pallas import tpu as pltpu
from jax.experimental.pallas import tpu_sc as plsc

NC, NS = 2, 16
NW = NC * NS
D = 64
CHUNK = 128


@functools.partial(jax.jit, static_argnames=("n_rows",))
def _gather_rows(idx2d, table, n_rows):
    n_chunks_total = idx2d.shape[0]
    n_chunks = n_chunks_total // NW
    rows_per_w = n_chunks * CHUNK

    mesh = plsc.VectorSubcoreMesh(
        core_axis_name="c", subcore_axis_name="s",
        num_cores=NC, num_subcores=NS)

    @functools.partial(
        pl.kernel,
        out_type=jax.ShapeDtypeStruct((n_rows, D), jnp.float32),
        mesh=mesh,
        scratch_types=[
            pltpu.VMEM((n_chunks, CHUNK), jnp.int32),
            pltpu.VMEM((CHUNK, D), jnp.float32),
            pltpu.SemaphoreType.DMA,
        ],
        compiler_params=pltpu.CompilerParams(use_tc_tiling_on_sc=False),
    )
    def k(idx_hbm, table_hbm, out_hbm, idx_v, rows_v, sem):
        wid = lax.axis_index("s") * NC + lax.axis_index("c")
        pltpu.sync_copy(idx_hbm.at[pl.ds(wid * n_chunks, n_chunks)], idx_v)
        row_base = wid * rows_per_w

        @pl.loop(0, n_chunks)
        def body(j):
            pltpu.async_copy(table_hbm.at[idx_v.at[j]], rows_v, sem).wait()
            pltpu.sync_copy(rows_v, out_hbm.at[pl.ds(row_base + j * CHUNK, CHUNK)])

    return k(idx2d, table)


def kernel(token_ids, embeddings):
    b, s = token_ids.shape
    n_rows = b * s
    idx2d = token_ids.astype(jnp.int32).reshape(n_rows // CHUNK, CHUNK)
    out = _gather_rows(idx2d, embeddings, n_rows)
    return out.reshape(b, s, D)


# speedup vs baseline: 1.8755x; 1.1127x over previous
import functools
import jax
import jax.numpy as jnp
from jax import lax
from jax.experimental import pallas as pl
from jax.experimental.pallas import tpu as pltpu
from jax.experimental.pallas import tpu_sc as plsc

NC, NS = 2, 16
NW = NC * NS
D = 64
CHUNK = 128


NBUF = 8


@functools.partial(jax.jit, static_argnames=("n_rows",))
def _gather_rows(idx2d, table, n_rows):
    n_chunks_total = idx2d.shape[0]
    n_chunks = n_chunks_total // NW
    rows_per_w = n_chunks * CHUNK

    mesh = plsc.VectorSubcoreMesh(
        core_axis_name="c", subcore_axis_name="s",
        num_cores=NC, num_subcores=NS)

    @functools.partial(
        pl.kernel,
        out_type=jax.ShapeDtypeStruct((n_rows, D), jnp.float32),
        mesh=mesh,
        scratch_types=[
            pltpu.VMEM((n_chunks, CHUNK), jnp.int32),
            pltpu.VMEM((NBUF, CHUNK, D), jnp.float32),
        ] + [pltpu.SemaphoreType.DMA] * NBUF,
        compiler_params=pltpu.CompilerParams(use_tc_tiling_on_sc=False),
    )
    def k(idx_hbm, table_hbm, out_hbm, idx_v, rows_v, *sems):
        wid = lax.axis_index("s") * NC + lax.axis_index("c")
        pltpu.sync_copy(idx_hbm.at[pl.ds(wid * n_chunks, n_chunks)], idx_v)
        row_base = wid * rows_per_w

        def start_gather(j, b):
            pltpu.async_copy(table_hbm.at[idx_v.at[j]], rows_v.at[b], sems[b])

        def wait_gather(b):
            pltpu.make_async_copy(
                table_hbm.at[idx_v.at[0]], rows_v.at[b], sems[b]).wait()

        def store(j, b):
            pltpu.sync_copy(
                rows_v.at[b], out_hbm.at[pl.ds(row_base + j * CHUNK, CHUNK)])

        for b in range(NBUF):
            start_gather(b, b)

        @pl.loop(0, n_chunks - NBUF, step=NBUF)
        def body(g):
            for b in range(NBUF):
                j = g + b
                wait_gather(b)
                store(j, b)
                start_gather(j + NBUF, b)

        for b in range(NBUF):
            wait_gather(b)
            store(n_chunks - NBUF + b, b)

    return k(idx2d, table)


def kernel(token_ids, embeddings):
    b, s = token_ids.shape
    n_rows = b * s
    idx2d = token_ids.astype(jnp.int32).reshape(n_rows // CHUNK, CHUNK)
    out = _gather_rows(idx2d, embeddings, n_rows)
    return out.reshape(b, s, D)
